# baseline (speedup 1.0000x reference)
import functools

import jax
import jax.numpy as jnp
from jax import lax
from jax.experimental import pallas as pl
from jax.experimental.pallas import tpu as pltpu
from jax.experimental.pallas import tpu_sc as plsc

_V = 1_000_000
_D = 64
_NJ = 200
_NI = 4096
_NC, _NSUB = 2, 16
_NW = _NC * _NSUB

_FULL_UNITS = _V // 128
_K1_BASE = _FULL_UNITS // _NW
_K1_REM = _FULL_UNITS % _NW
_TAIL_COLS = _V - _FULL_UNITS * 128

_UNITS2 = _NJ * 8
_UPW2 = _UNITS2 // _NW

_mesh = plsc.VectorSubcoreMesh(core_axis_name="c", subcore_axis_name="s")
_params = pltpu.CompilerParams(use_tc_tiling_on_sc=True, needs_layout_passes=False)

def _iota16():
    return jax.lax.broadcasted_iota(jnp.int32, (16,), 0)


@functools.partial(
    pl.kernel,
    mesh=_mesh,
    out_type=jax.ShapeDtypeStruct((_V // 2, 128), jnp.float32),
    scratch_types=[
        pltpu.VMEM((2, 64, 128), jnp.float32),
        pltpu.VMEM((2, 64, 128), jnp.float32),
        pltpu.VMEM((64, 64), jnp.float32),
        [pltpu.SemaphoreType.DMA] * 2,
        [pltpu.SemaphoreType.DMA] * 2,
    ],
    compiler_params=_params,
)
def _transpose_table(w_hbm, scr, tbuf, rstage, tbuf_t, isems, osems):
    wid = lax.axis_index("s") * _NC + lax.axis_index("c")
    iota16 = _iota16()
    cnt = _K1_BASE + jnp.where(wid < _K1_REM, 1, 0)
    start = _K1_BASE * wid + jnp.minimum(wid, _K1_REM)

    def load(u, b):
        pltpu.async_copy(w_hbm.at[:, pl.ds((start + u) * 128, 128)], tbuf.at[b], isems[b])

    def wait_load(b):
        pltpu.make_async_copy(w_hbm.at[:, pl.ds(0, 128)], tbuf.at[b], isems[b]).wait()

    def put(u, b):
        pltpu.async_copy(rstage.at[b], scr.at[pl.ds((start + u) * 64, 64)], osems[b])

    def wait_put(b):
        pltpu.make_async_copy(rstage.at[b], scr.at[pl.ds(0, 64)], osems[b]).wait()

    def trans(src, dst, np_rows):
        def body(p, carry):
            c0 = jnp.broadcast_to(2 * p, (16,))
            c1 = c0 + 1
            for m in range(8):
                rows = iota16 + (m % 4) * 16
                cols = c0 if m < 4 else c1
                dst[p, pl.ds(m * 16, 16)] = plsc.load_gather(src, [rows, cols])
            return carry

        lax.fori_loop(0, np_rows, body, 0)

    load(0, 0)

    def unit_pair(t, carry):
        for b in range(2):
            u = 2 * t + b

            @pl.when(u < cnt)
            def _(u=u, b=b):
                @pl.when(u + 1 < cnt)
                def _():
                    load(u + 1, 1 - b)

                wait_load(b)

                @pl.when(u >= 2)
                def _():
                    wait_put(b)

                trans(tbuf.at[b], rstage.at[b], 64)
                put(u, b)
        return carry

    lax.fori_loop(0, (cnt + 1) // 2, unit_pair, 0)

    wait_put(0)
    wait_put(1)

    @pl.when(wid == _NW - 1)
    def _():
        pltpu.sync_copy(w_hbm.at[:, pl.ds(_FULL_UNITS * 128, _TAIL_COLS)], tbuf_t)

        def body(p, carry):
            c0 = jnp.broadcast_to(2 * p, (16,))
            c1 = c0 + 1
            for m in range(8):
                rows = iota16 + (m % 4) * 16
                cols = c0 if m < 4 else c1
                rstage[0, p, pl.ds(m * 16, 16)] = plsc.load_gather(
                    tbuf_t, [rows, cols]
                )
            return carry

        lax.fori_loop(0, _TAIL_COLS // 2, body, 0)
        pltpu.sync_copy(
            rstage.at[0, pl.ds(0, _TAIL_COLS // 2)],
            scr.at[pl.ds(_FULL_UNITS * 64, _TAIL_COLS // 2)],
        )


@functools.partial(
    pl.kernel,
    mesh=_mesh,
    out_type=jax.ShapeDtypeStruct((_NJ, 8, 32, 8, 128), jnp.float32),
    scratch_types=[
        pltpu.VMEM((8, 512), jnp.int32),
        pltpu.VMEM((4, 128), jnp.int32),
        pltpu.VMEM((2, 128, 128), jnp.float32),
        pltpu.VMEM((2, 8, 4, 8, 128), jnp.float32),
        [pltpu.SemaphoreType.DMA] * 2,
        [pltpu.SemaphoreType.DMA] * 2,
    ],
    compiler_params=_params,
)
def _gather_ids(ids_hbm, scr_hbm, out_hbm, idtile, idxb, G, slab, gsems, osems):
    wid = lax.axis_index("s") * _NC + lax.axis_index("c")
    u0 = wid * _UPW2
    iota16 = _iota16()

    def gather(cc, b):
        pltpu.async_copy(scr_hbm.at[idxb.at[cc]], G.at[b], gsems[b])

    def wait_gather(b):
        pltpu.make_async_copy(scr_hbm.at[idxb.at[0]], G.at[b], gsems[b]).wait()

    def wait_slab(b):
        for kb in range(8):
            pltpu.make_async_copy(
                slab.at[b, kb], out_hbm.at[0, kb, pl.ds(0, 4)], osems[b]
            ).wait()

    def unit(uu, sb):
        u = u0 + uu
        j = u // 8
        ibB = u % 8
        jb = j // 8
        jr = j % 8
        pltpu.sync_copy(
            ids_hbm.at[pl.ds(jb * 8, 8), pl.ds(ibB * 512, 512)], idtile
        )

        def mkidx(m, c2):
            v = idtile[jr, pl.ds(m * 16, 16)]
            idxb[m // 8, pl.ds((m % 8) * 16, 16)] = jax.lax.shift_right_logical(v, 1)
            return c2

        lax.fori_loop(0, 32, mkidx, 0)

        @pl.when(uu >= 2)
        def _():
            wait_slab(sb)

        gather(0, 0)
        for cc in range(4):
            if cc + 1 < 4:
                gather(cc + 1, (cc + 1) % 2)
            wait_gather(cc % 2)
            Gb = G.at[cc % 2]
            hv = [
                jax.lax.shift_left(idtile[jr, pl.ds((cc * 8 + g) * 16, 16)] & 1, 6)
                for g in range(8)
            ]
            rows = [iota16 + g * 16 for g in range(8)]

            def sel(k, c3):
                for g in range(8):
                    v = plsc.load_gather(Gb, [rows[g], hv[g] + k])
                    slab[sb, k // 8, cc, k % 8, pl.ds(g * 16, 16)] = v
                return c3

            lax.fori_loop(0, 64, sel, 0)

        for kb in range(8):
            pltpu.async_copy(
                slab.at[sb, kb], out_hbm.at[j, kb, pl.ds(ibB * 4, 4)], osems[sb]
            )

    def unit_pair(t, carry):
        for b in range(2):
            unit(2 * t + b, b)
        return carry

    lax.fori_loop(0, _UPW2 // 2, unit_pair, 0)
    wait_slab(0)
    wait_slab(1)


def kernel(ids, weight):
    ids_t = ids.T.astype(jnp.int32)
    w_t = weight.T
    scr = _transpose_table(w_t)
    out5 = _gather_ids(ids_t, scr)
    return out5.transpose(2, 4, 0, 1, 3).reshape(_NI, _NJ, _D)


# speedup vs baseline: 1.2068x; 1.2068x over previous
import functools

import jax
import jax.numpy as jnp
from jax import lax
from jax.experimental import pallas as pl
from jax.experimental.pallas import tpu as pltpu
from jax.experimental.pallas import tpu_sc as plsc

_V = 1_000_000
_D = 64
_NJ = 200
_NI = 4096
_NC, _NSUB = 2, 16
_NW = _NC * _NSUB

_FULL_UNITS = _V // 128
_K1_BASE = _FULL_UNITS // _NW
_K1_REM = _FULL_UNITS % _NW
_TAIL_COLS = _V - _FULL_UNITS * 128

_UNITS2 = _NJ * 16
_UPW2 = _UNITS2 // _NW

_SK = 65

_mesh = plsc.VectorSubcoreMesh(core_axis_name="c", subcore_axis_name="s")
_params = pltpu.CompilerParams(use_tc_tiling_on_sc=True, needs_layout_passes=False)


def _iota16():
    return jax.lax.broadcasted_iota(jnp.int32, (16,), 0)


@functools.partial(
    pl.kernel,
    mesh=_mesh,
    out_type=jax.ShapeDtypeStruct((_V // 2, 128), jnp.float32),
    scratch_types=[
        pltpu.VMEM((2, 64, 128), jnp.float32),
        pltpu.VMEM((2, 64, 128), jnp.float32),
        pltpu.VMEM((128 * _SK,), jnp.float32),
        pltpu.VMEM((64, 64), jnp.float32),
        [pltpu.SemaphoreType.DMA] * 2,
        [pltpu.SemaphoreType.DMA] * 2,
    ],
    compiler_params=_params,
)
def _transpose_table(w_hbm, scr, tbuf, rstage, S, tbuf_t, isems, osems):
    wid = lax.axis_index("s") * _NC + lax.axis_index("c")
    iota16 = _iota16()
    cnt = _K1_BASE + jnp.where(wid < _K1_REM, 1, 0)
    start = _K1_BASE * wid + jnp.minimum(wid, _K1_REM)
    qsk = [(iota16 + 16 * m) * _SK for m in range(8)]

    def load(u, b):
        pltpu.async_copy(w_hbm.at[:, pl.ds((start + u) * 128, 128)], tbuf.at[b], isems[b])

    def wait_load(b):
        pltpu.make_async_copy(w_hbm.at[:, pl.ds(0, 128)], tbuf.at[b], isems[b]).wait()

    def put(u, b):
        pltpu.async_copy(rstage.at[b], scr.at[pl.ds((start + u) * 64, 64)], osems[b])

    def wait_put(b):
        pltpu.make_async_copy(rstage.at[b], scr.at[pl.ds(0, 64)], osems[b]).wait()

    def trans(src, dst, ncols, npairs):
        nm = ncols // 16

        def stage_a(k, carry):
            for m in range(nm):
                v = src[k, pl.ds(m * 16, 16)]
                plsc.store_scatter(S, [qsk[m] + k], v)
            return carry

        lax.fori_loop(0, 64, stage_a, 0)

        def stage_b(p, carry):
            for m in range(8):
                dst[p, pl.ds(m * 16, 16)] = S[
                    pl.ds((2 * p + m // 4) * _SK + (m % 4) * 16, 16)
                ]
            return carry

        lax.fori_loop(0, npairs, stage_b, 0)

    load(0, 0)

    def unit_pair(t, carry):
        for b in range(2):
            u = 2 * t + b

            @pl.when(u < cnt)
            def _(u=u, b=b):
                @pl.when(u + 1 < cnt)
                def _():
                    load(u + 1, 1 - b)

                wait_load(b)

                @pl.when(u >= 2)
                def _():
                    wait_put(b)

                trans(tbuf.at[b], rstage.at[b], 128, 64)
                put(u, b)
        return carry

    lax.fori_loop(0, (cnt + 1) // 2, unit_pair, 0)
    wait_put(0)
    wait_put(1)

    @pl.when(wid == _NW - 1)
    def _():
        pltpu.sync_copy(w_hbm.at[:, pl.ds(_FULL_UNITS * 128, _TAIL_COLS)], tbuf_t)
        trans(tbuf_t, rstage.at[0], _TAIL_COLS, _TAIL_COLS // 2)
        pltpu.sync_copy(
            rstage.at[0, pl.ds(0, _TAIL_COLS // 2)],
            scr.at[pl.ds(_FULL_UNITS * 64, _TAIL_COLS // 2)],
        )


@functools.partial(
    pl.kernel,
    mesh=_mesh,
    out_type=jax.ShapeDtypeStruct((_NJ, 8, 32, 8, 128), jnp.float32),
    scratch_types=[
        pltpu.VMEM((8, 256), jnp.int32),
        pltpu.VMEM((2, 128), jnp.int32),
        pltpu.VMEM((2, 128, 130), jnp.float32),
        pltpu.VMEM((2, 8, 2, 8, 128), jnp.float32),
        [pltpu.SemaphoreType.DMA] * 2,
        [pltpu.SemaphoreType.DMA] * 2,
    ],
    compiler_params=_params,
)
def _gather_ids(ids_hbm, scr_hbm, out_hbm, idtile, idxb, G, slab, gsems, osems):
    wid = lax.axis_index("s") * _NC + lax.axis_index("c")
    iota16 = _iota16()
    u0 = wid * _UPW2
    qrow = [iota16 + 16 * g for g in range(8)]

    def gather(cc, b):
        pltpu.async_copy(
            scr_hbm.at[idxb.at[cc]], G.at[b, :, pl.ds(0, 128)], gsems[b]
        )

    def wait_gather(b):
        pltpu.make_async_copy(
            scr_hbm.at[idxb.at[0]], G.at[b, :, pl.ds(0, 128)], gsems[b]
        ).wait()

    def wait_slab(b):
        for kb in range(8):
            pltpu.make_async_copy(
                slab.at[b, kb], out_hbm.at[0, kb, pl.ds(0, 2)], osems[b]
            ).wait()

    def unit(uu, sb):
        u = u0 + uu
        j = u // 16
        ibB = u % 16
        jb = j // 8
        jr = j % 8
        pltpu.sync_copy(
            ids_hbm.at[pl.ds(jb * 8, 8), pl.ds(ibB * 256, 256)], idtile
        )

        def mkidx(m, c2):
            v = idtile[jr, pl.ds(m * 16, 16)]
            idxb[m // 8, pl.ds((m % 8) * 16, 16)] = jax.lax.shift_right_logical(v, 1)
            return c2

        lax.fori_loop(0, 16, mkidx, 0)

        @pl.when(uu >= 2)
        def _():
            wait_slab(sb)

        gather(0, 0)
        for cc in range(2):
            if cc + 1 < 2:
                gather(cc + 1, (cc + 1) % 2)
            wait_gather(cc % 2)
            Gb = G.at[cc % 2]

            hv = []
            for g in range(8):
                idv = idtile[jr, pl.ds(cc * 128 + g * 16, 16)]
                hv.append((idv & 1) * 64)

            def stage_b(k, c4):
                for g in range(8):
                    v = plsc.load_gather(Gb, [qrow[g], hv[g] + k])
                    slab[sb, k // 8, cc, k % 8, pl.ds(g * 16, 16)] = v
                return c4

            lax.fori_loop(0, 64, stage_b, 0)

        for kb in range(8):
            pltpu.async_copy(
                slab.at[sb, kb], out_hbm.at[j, kb, pl.ds(ibB * 2, 2)], osems[sb]
            )

    def unit_pair(t, carry):
        for b in range(2):
            unit(2 * t + b, b)
        return carry

    lax.fori_loop(0, _UPW2 // 2, unit_pair, 0)
    wait_slab(0)
    wait_slab(1)


def kernel(ids, weight):
    ids_t = ids.T.astype(jnp.int32)
    w_t = weight.T
    scr = _transpose_table(w_t)
    out5 = _gather_ids(ids_t, scr)
    return out5.transpose(2, 4, 0, 1, 3).reshape(_NI, _NJ, _D)


# speedup vs baseline: 1.4582x; 1.2083x over previous
import functools

import jax
import jax.numpy as jnp
from jax import lax
from jax.experimental import pallas as pl
from jax.experimental.pallas import tpu as pltpu
from jax.experimental.pallas import tpu_sc as plsc

_V = 1_000_000
_D = 64
_NJ = 200
_NI = 4096
_NC, _NSUB = 2, 16
_NW = _NC * _NSUB

_TCB = 512
_TCG = -(-_V // _TCB)

_UNITS = _NJ * (_NI // 128)
_UPW = _UNITS // _NW
_NB = 4
_LA = 2

_mesh = plsc.VectorSubcoreMesh(core_axis_name="c", subcore_axis_name="s")
_params = pltpu.CompilerParams(use_tc_tiling_on_sc=True, needs_layout_passes=False)


def _tc_transpose(w_t):
    def body(in_ref, out_ref):
        xt = jnp.transpose(in_ref[...])
        out_ref[...] = jnp.concatenate([xt, xt], axis=1)

    return pl.pallas_call(
        body,
        grid=(_TCG,),
        in_specs=[pl.BlockSpec((64, _TCB), lambda i: (0, i))],
        out_specs=pl.BlockSpec((_TCB, 128), lambda i: (i, 0)),
        out_shape=jax.ShapeDtypeStruct((_V, 128), jnp.float32),
    )(w_t)


@functools.partial(
    pl.kernel,
    mesh=_mesh,
    out_type=jax.ShapeDtypeStruct((_NJ, _NI, 128), jnp.float32),
    scratch_types=[
        pltpu.VMEM((_NB, 8, 128), jnp.int32),
        pltpu.VMEM((_NB, 128, 128), jnp.float32),
        [pltpu.SemaphoreType.DMA] * _NB,
        [pltpu.SemaphoreType.DMA] * _NB,
    ],
    compiler_params=_params,
)
def _gather_ids(ids_hbm, scr_hbm, out_hbm, idt, G, gsems, osems):
    wid = lax.axis_index("s") * _NC + lax.axis_index("c")
    u0 = wid * _UPW

    def coords(u):
        g = u0 + u
        j = g // 32
        ib = g % 32
        return j, j // 8, j % 8, ib

    def load_ids(u, b):
        _, jb, _, ib = coords(u)
        pltpu.sync_copy(
            ids_hbm.at[pl.ds(jb * 8, 8), pl.ds(ib * 128, 128)], idt.at[b]
        )

    def gather(u, b):
        _, _, jr, _ = coords(u)
        pltpu.async_copy(scr_hbm.at[idt.at[b, jr]], G.at[b], gsems[b])

    def wait_gather(b):
        pltpu.make_async_copy(scr_hbm.at[idt.at[0, 0]], G.at[b], gsems[b]).wait()

    def put(u, b):
        j, _, _, ib = coords(u)
        pltpu.async_copy(
            G.at[b], out_hbm.at[j, pl.ds(ib * 128, 128)], osems[b]
        )

    def wait_put(b):
        pltpu.make_async_copy(
            G.at[b], out_hbm.at[0, pl.ds(0, 128)], osems[b]
        ).wait()

    for b in range(_LA):
        load_ids(b, b)
        gather(b, b)

    def quad(t, carry):
        for s in range(_NB):
            u = _NB * t + s
            b2 = (s + _LA) % _NB

            @pl.when(u + _LA < _UPW)
            def _(u=u, b2=b2):
                @pl.when(u + _LA >= _NB)
                def _():
                    wait_put(b2)

                load_ids(u + _LA, b2)
                gather(u + _LA, b2)

            wait_gather(s)
            put(u, s)
        return carry

    lax.fori_loop(0, _UPW // _NB, quad, 0)
    for b in range(_NB):
        wait_put(b)


def kernel(ids, weight):
    ids_t = ids.T.astype(jnp.int32)
    w_t = weight.T
    scr = _tc_transpose(w_t)
    out3 = _gather_ids(ids_t, scr)
    return out3[:, :, : _D].transpose(1, 0, 2)


# speedup vs baseline: 2.0609x; 1.4133x over previous
import functools

import jax
import jax.numpy as jnp
from jax import lax
from jax.experimental import pallas as pl
from jax.experimental.pallas import tpu as pltpu
from jax.experimental.pallas import tpu_sc as plsc

_V = 1_000_000
_D = 64
_NJ = 200
_NI = 4096
_NC, _NSUB = 2, 16
_NW = _NC * _NSUB

_TCB = 512
_TCG = -(-_V // _TCB)

_UNITS = _NJ * (_NI // 128)
_UPW = _UNITS // _NW
_NB = 4
_LA = 2

_mesh = plsc.VectorSubcoreMesh(core_axis_name="c", subcore_axis_name="s")
_params = pltpu.CompilerParams(use_tc_tiling_on_sc=True, needs_layout_passes=False)


def _tc_transpose(w_t):
    def body(in_ref, out_ref):
        xt = jnp.transpose(in_ref[...])
        out_ref[...] = jnp.concatenate([xt, xt], axis=1)

    return pl.pallas_call(
        body,
        grid=(_TCG,),
        in_specs=[pl.BlockSpec((64, _TCB), lambda i: (0, i))],
        out_specs=pl.BlockSpec((_TCB, 128), lambda i: (i, 0)),
        out_shape=jax.ShapeDtypeStruct((_V, 128), jnp.float32),
    )(w_t)


@functools.partial(
    pl.kernel,
    mesh=_mesh,
    out_type=jax.ShapeDtypeStruct((_NJ, _NI, 128), jnp.float32),
    scratch_types=[
        pltpu.VMEM((_NB, 8, 128), jnp.int32),
        pltpu.VMEM((_NB, 128, 128), jnp.float32),
        [pltpu.SemaphoreType.DMA] * _NB,
        [pltpu.SemaphoreType.DMA] * _NB,
    ],
    compiler_params=_params,
)
def _gather_ids(ids_hbm, scr_hbm, out_hbm, idt, G, gsems, osems):
    wid = lax.axis_index("s") * _NC + lax.axis_index("c")
    u0 = wid * _UPW

    def coords(u):
        g = u0 + u
        j = g // 32
        ib = g % 32
        return j, j // 8, j % 8, ib

    def load_ids(u, b):
        _, jb, _, ib = coords(u)
        pltpu.sync_copy(
            ids_hbm.at[pl.ds(jb * 8, 8), pl.ds(ib * 128, 128)], idt.at[b]
        )

    def gather(u, b):
        _, _, jr, _ = coords(u)
        pltpu.async_copy(scr_hbm.at[idt.at[b, jr]], G.at[b], gsems[b])

    def wait_gather(b):
        pltpu.make_async_copy(scr_hbm.at[idt.at[0, 0]], G.at[b], gsems[b]).wait()

    def put(u, b):
        j, _, _, ib = coords(u)
        pltpu.async_copy(
            G.at[b], out_hbm.at[j, pl.ds(ib * 128, 128)], osems[b]
        )

    def wait_put(b):
        pltpu.make_async_copy(
            G.at[b], out_hbm.at[0, pl.ds(0, 128)], osems[b]
        ).wait()

    for b in range(_LA):
        load_ids(b, b)
        gather(b, b)

    def quad(t, carry):
        for s in range(_NB):
            u = _NB * t + s
            b2 = (s + _LA) % _NB

            @pl.when(u + _LA < _UPW)
            def _(u=u, b2=b2):
                @pl.when(u + _LA >= _NB)
                def _():
                    wait_put(b2)

                load_ids(u + _LA, b2)
                gather(u + _LA, b2)

            wait_gather(s)
            put(u, s)
        return carry

    lax.fori_loop(0, _UPW // _NB, quad, 0)
    for b in range(_NB):
        wait_put(b)


def kernel(ids, weight):
    ids_t = ids.T.astype(jnp.int32)
    scr = jnp.concatenate([weight, weight], axis=1)
    out3 = _gather_ids(ids_t, scr)
    return out3[:, :, : _D].transpose(1, 0, 2)


# speedup vs baseline: 2.3282x; 1.1297x over previous
import functools

import jax
import jax.numpy as jnp
from jax import lax
from jax.experimental import pallas as pl
from jax.experimental.pallas import tpu as pltpu
from jax.experimental.pallas import tpu_sc as plsc

_NB, _NS = 4096, 200
_B = _NB * _NS
_D = 64
_NC, _NSUB = 2, 16
_NW = _NC * _NSUB
_BPW = _B // _NW
_CH = 128
_NCH = _BPW // _CH
_NBUF = 8
_LA = 4

_mesh = plsc.VectorSubcoreMesh(core_axis_name="c", subcore_axis_name="s")


@functools.partial(
    pl.kernel,
    mesh=_mesh,
    out_type=jax.ShapeDtypeStruct((_B, _D), jnp.float32),
    scratch_types=[
        pltpu.VMEM((_NCH, _CH), jnp.int32),
        pltpu.VMEM((_NBUF, _CH, _D), jnp.float32),
        [pltpu.SemaphoreType.DMA] * _NBUF,
        [pltpu.SemaphoreType.DMA] * _NBUF,
    ],
    compiler_params=pltpu.CompilerParams(use_tc_tiling_on_sc=False),
)
def _emb_lookup(ids_hbm, w_hbm, out_hbm, idx_v, rows_v, gsems, psems):
    wid = lax.axis_index("s") * _NC + lax.axis_index("c")
    base = wid * _BPW
    pltpu.sync_copy(ids_hbm.at[wid], idx_v)

    def gather(g, b):
        pltpu.async_copy(w_hbm.at[idx_v.at[g]], rows_v.at[b], gsems[b])

    def put(g, b):
        pltpu.async_copy(
            rows_v.at[b], out_hbm.at[pl.ds(base + g * _CH, _CH)], psems[b]
        )

    def wait_gather(b):
        pltpu.make_async_copy(w_hbm.at[idx_v.at[0]], rows_v.at[b], gsems[b]).wait()

    def wait_put(b):
        pltpu.make_async_copy(
            rows_v.at[b], out_hbm.at[pl.ds(base, _CH)], psems[b]
        ).wait()

    for b in range(_LA):
        gather(b, b)

    def outer(i, carry):
        g0 = i * _NBUF
        for b in range(_NBUF):
            g = g0 + b
            wait_gather(b)
            put(g, b)
            b2 = (b + _LA) % _NBUF

            @pl.when(g >= _NBUF - _LA)
            def _():
                wait_put(b2)

            @pl.when(g + _LA < _NCH)
            def _():
                gather(g + _LA, b2)
        return carry

    lax.fori_loop(0, _NCH // _NBUF, outer, 0)

    for g in range(_NCH - _LA, _NCH):
        wait_put(g % _NBUF)


def kernel(ids, weight):
    flat_ids = ids.reshape(_NW, _NCH, _CH).astype(jnp.int32)
    out = _emb_lookup(flat_ids, weight)
    return out.reshape(_NB, _NS, _D)
